# baseline (speedup 1.0000x reference)
import functools

import jax
import jax.numpy as jnp
from jax import lax
from jax.experimental import pallas as pl
from jax.experimental.pallas import tpu as pltpu
from jax.experimental.pallas import tpu_sc as plsc

KK = 1024
DD = 64
B0, B1 = 8, 576
ROWS = B0 * B1
NCH = 8
BETA = 0.25
LOSS_SCALE = (1.0 + BETA) / (ROWS * DD)

NC, NS = 2, 16
NW = NC * NS
BPW = ROWS // NW
WPB = B1 // BPW
DP = 128


def _dist_argmin_body(lat_ref, cb_ref, idx_ref, loss_ref):
    lat = lat_ref[...]
    cb = cb_ref[...]
    cn = jnp.sum(cb * cb, axis=1)
    rn = jnp.sum(lat * lat, axis=1)
    dots2 = lax.dot_general(lat, cb + cb, (((1,), (1,)), ((), ())),
                            preferred_element_type=jnp.float32)
    dist = (rn[:, None] + cn[None, :]) - dots2

    minval = jnp.min(dist, axis=1)
    hit = dist == minval[:, None]
    iota = lax.broadcasted_iota(jnp.int32, (ROWS, KK), 1).astype(jnp.float32)
    idxf = jnp.min(jnp.where(hit, iota, float(2 * KK)), axis=1)
    idx_ref[...] = idxf.astype(jnp.int32)
    loss_ref[...] = (jnp.sum(minval) * LOSS_SCALE).reshape(1, 1)


_dist_argmin = pl.pallas_call(
    _dist_argmin_body,
    in_specs=[
        pl.BlockSpec((ROWS, DD), lambda: (0, 0)),
        pl.BlockSpec((KK, DD), lambda: (0, 0)),
    ],
    out_specs=[
        pl.BlockSpec((ROWS,), lambda: (0,)),
        pl.BlockSpec((1, 1), lambda: (0, 0)),
    ],
    out_shape=[
        jax.ShapeDtypeStruct((ROWS,), jnp.int32),
        jax.ShapeDtypeStruct((1, 1), jnp.float32),
    ],
)


def _sc_gather_body(cb_hbm, idx_hbm, out_hbm, idx_v, rows_v, sem):
    wid = lax.axis_index("s") * NC + lax.axis_index("c")
    base = wid * BPW
    pltpu.sync_copy(idx_hbm.at[pl.ds(base, BPW)], idx_v)
    pltpu.async_copy(cb_hbm.at[idx_v], rows_v, sem).wait()
    b = wid // WPB
    s0 = (wid % WPB) * BPW
    pltpu.sync_copy(rows_v, out_hbm.at[b, pl.ds(s0, BPW), :])


@functools.cache
def _make_sc_gather():
    return functools.partial(
        pl.kernel,
        out_type=jax.ShapeDtypeStruct((B0, B1, DD), jnp.float32),
        mesh=plsc.VectorSubcoreMesh(core_axis_name="c", subcore_axis_name="s"),
        scratch_types=[
            pltpu.VMEM((BPW,), jnp.int32),
            pltpu.VMEM((BPW, DD), jnp.float32),
            pltpu.SemaphoreType.DMA,
        ],
        compiler_params=pltpu.CompilerParams(use_tc_tiling_on_sc=False,
                                             skip_device_barrier=True),
    )(_sc_gather_body)


def kernel(latents, codebook):
    idx, loss = _dist_argmin(latents.reshape(ROWS, DD), codebook)
    quantized = _make_sc_gather()(codebook, idx)
    return quantized, loss[0, 0]


# speedup vs baseline: 1.0330x; 1.0330x over previous
import functools

import jax
import jax.numpy as jnp
from jax import lax
from jax.experimental import pallas as pl
from jax.experimental.pallas import tpu as pltpu
from jax.experimental.pallas import tpu_sc as plsc

KK = 1024
DD = 64
B0, B1 = 8, 576
ROWS = B0 * B1
NCH = 8
BETA = 0.25
LOSS_SCALE = (1.0 + BETA) / (ROWS * DD)

NC, NS = 2, 16
NW = NC * NS
BPW = ROWS // NW
WPB = B1 // BPW
DP = 128


def _dist_argmin_body(lat_ref, cb_ref, idx_ref, loss_ref):
    lat = lat_ref[...]
    cb = cb_ref[...]
    cn = jnp.sum(cb * cb, axis=1)
    rn = jnp.sum(lat * lat, axis=1)
    dots2 = lax.dot_general(lat, cb + cb, (((1,), (1,)), ((), ())),
                            preferred_element_type=jnp.float32)
    dist = (rn[:, None] + cn[None, :]) - dots2

    minval = jnp.min(dist, axis=1)
    hit = dist == minval[:, None]
    iota = lax.broadcasted_iota(jnp.int32, (ROWS, KK), 1)
    idx = jnp.min(jnp.where(hit, iota, KK), axis=1)
    idx_ref[...] = idx
    loss_ref[...] = (jnp.sum(minval) * LOSS_SCALE).reshape(1, 1)


_dist_argmin = pl.pallas_call(
    _dist_argmin_body,
    in_specs=[
        pl.BlockSpec((ROWS, DD), lambda: (0, 0)),
        pl.BlockSpec((KK, DD), lambda: (0, 0)),
    ],
    out_specs=[
        pl.BlockSpec((ROWS,), lambda: (0,)),
        pl.BlockSpec((1, 1), lambda: (0, 0)),
    ],
    out_shape=[
        jax.ShapeDtypeStruct((ROWS,), jnp.int32),
        jax.ShapeDtypeStruct((1, 1), jnp.float32),
    ],
)


def _sc_gather_body(cb_hbm, idx_hbm, out_hbm, idx_v, rows_v, sem):
    wid = lax.axis_index("s") * NC + lax.axis_index("c")
    base = wid * BPW
    pltpu.sync_copy(idx_hbm.at[pl.ds(base, BPW)], idx_v)
    pltpu.async_copy(cb_hbm.at[idx_v], rows_v, sem).wait()
    b = wid // WPB
    s0 = (wid % WPB) * BPW
    pltpu.sync_copy(rows_v, out_hbm.at[b, pl.ds(s0, BPW), :])


@functools.cache
def _make_sc_gather():
    return functools.partial(
        pl.kernel,
        out_type=jax.ShapeDtypeStruct((B0, B1, DD), jnp.float32),
        mesh=plsc.VectorSubcoreMesh(core_axis_name="c", subcore_axis_name="s"),
        scratch_types=[
            pltpu.VMEM((BPW,), jnp.int32),
            pltpu.VMEM((BPW, DD), jnp.float32),
            pltpu.SemaphoreType.DMA,
        ],
        compiler_params=pltpu.CompilerParams(use_tc_tiling_on_sc=False,
                                             skip_device_barrier=True),
    )(_sc_gather_body)


def kernel(latents, codebook):
    idx, loss = _dist_argmin(latents.reshape(ROWS, DD), codebook)
    quantized = _make_sc_gather()(codebook, idx)
    return quantized, loss[0, 0]


# speedup vs baseline: 1.0535x; 1.0198x over previous
import functools

import jax
import jax.numpy as jnp
from jax import lax
from jax.experimental import pallas as pl
from jax.experimental.pallas import tpu as pltpu
from jax.experimental.pallas import tpu_sc as plsc

KK = 1024
DD = 64
B0, B1 = 8, 576
ROWS = B0 * B1
NCH = 8
BETA = 0.25
LOSS_SCALE = (1.0 + BETA) / (ROWS * DD)

NC, NS = 1, 16
NW = NC * NS
BPW = ROWS // NW
WPB = B1 // BPW
DP = 128


def _dist_argmin_body(lat_ref, cb_ref, idx_ref, loss_ref):
    lat = lat_ref[...]
    cb = cb_ref[...]
    cn = jnp.sum(cb * cb, axis=1)
    rn = jnp.sum(lat * lat, axis=1)
    dots2 = lax.dot_general(lat, cb + cb, (((1,), (1,)), ((), ())),
                            preferred_element_type=jnp.float32)
    dist = (rn[:, None] + cn[None, :]) - dots2

    minval = jnp.min(dist, axis=1)
    hit = dist == minval[:, None]
    iota = lax.broadcasted_iota(jnp.int32, (ROWS, KK), 1)
    idx = jnp.min(jnp.where(hit, iota, KK), axis=1)
    idx_ref[...] = idx
    loss_ref[...] = (jnp.sum(minval) * LOSS_SCALE).reshape(1, 1)


_dist_argmin = pl.pallas_call(
    _dist_argmin_body,
    in_specs=[
        pl.BlockSpec((ROWS, DD), lambda: (0, 0)),
        pl.BlockSpec((KK, DD), lambda: (0, 0)),
    ],
    out_specs=[
        pl.BlockSpec((ROWS,), lambda: (0,)),
        pl.BlockSpec((1, 1), lambda: (0, 0)),
    ],
    out_shape=[
        jax.ShapeDtypeStruct((ROWS,), jnp.int32),
        jax.ShapeDtypeStruct((1, 1), jnp.float32),
    ],
)


def _sc_gather_body(cb_hbm, idx_hbm, out_hbm, idx_v, rows_v, sem):
    wid = lax.axis_index("s") * NC + lax.axis_index("c")
    base = wid * BPW
    pltpu.sync_copy(idx_hbm.at[pl.ds(base, BPW)], idx_v)
    pltpu.async_copy(cb_hbm.at[idx_v], rows_v, sem).wait()
    b = wid // WPB
    s0 = (wid % WPB) * BPW
    pltpu.sync_copy(rows_v, out_hbm.at[b, pl.ds(s0, BPW), :])


@functools.cache
def _make_sc_gather():
    return functools.partial(
        pl.kernel,
        out_type=jax.ShapeDtypeStruct((B0, B1, DD), jnp.float32),
        mesh=plsc.VectorSubcoreMesh(core_axis_name="c", subcore_axis_name="s", num_cores=1),
        scratch_types=[
            pltpu.VMEM((BPW,), jnp.int32),
            pltpu.VMEM((BPW, DD), jnp.float32),
            pltpu.SemaphoreType.DMA,
        ],
        compiler_params=pltpu.CompilerParams(use_tc_tiling_on_sc=False,
                                             skip_device_barrier=True),
    )(_sc_gather_body)


def kernel(latents, codebook):
    idx, loss = _dist_argmin(latents.reshape(ROWS, DD), codebook)
    quantized = _make_sc_gather()(codebook, idx)
    return quantized, loss[0, 0]
